# baseline (speedup 1.0000x reference)
import functools

import jax
import jax.numpy as jnp
from jax import lax
from jax.experimental import pallas as pl
from jax.experimental.pallas import tpu as pltpu
from jax.experimental.pallas import tpu_sc as plsc

B = 1024
D = 512
C = 1000
K = 5
BLK = 128
NBLK = B // BLK

NW = 32
RPW = B // NW
L = 16
NCH = D // L

_sc_mesh = plsc.VectorSubcoreMesh(core_axis_name="c", subcore_axis_name="s")


@functools.partial(
    pl.kernel,
    out_type=jax.ShapeDtypeStruct((8, B), jnp.int32),
    mesh=_sc_mesh,
    compiler_params=pltpu.CompilerParams(needs_layout_passes=False),
    scratch_types=[
        pltpu.VMEM((RPW * D,), jnp.float32),
        pltpu.VMEM((RPW,), jnp.int32),
        pltpu.VMEM((RPW,), jnp.int32),
    ],
)
def _sc_keys(feat_hbm, out_hbm, feat_l, ka_l, kb_l):
    wid = lax.axis_index("s") * 2 + lax.axis_index("c")
    base = wid * RPW
    pltpu.sync_copy(feat_hbm.at[pl.ds(base * D, RPW * D)], feat_l)

    iota = lax.broadcasted_iota(jnp.int32, (L,), 0)
    mask0 = iota == 0
    neg_inf = jnp.full((L,), -jnp.inf, jnp.float32)

    def row_body(r, _):
        row0 = r * D
        idxs = []
        for _t in range(K):
            def chunk_body(c, carry):
                lane_max, lane_chunk = carry
                chunk = feat_l[pl.ds(row0 + c * L, L)]
                gt = chunk > lane_max
                lane_max = jnp.where(gt, chunk, lane_max)
                lane_chunk = jnp.where(gt, jnp.broadcast_to(c, (L,)),
                                       lane_chunk)
                return lane_max, lane_chunk

            lane_max, lane_chunk = lax.fori_loop(
                0, NCH, chunk_body,
                (jnp.full((L,), -jnp.inf, jnp.float32),
                 jnp.zeros((L,), jnp.int32)))
            gm = plsc.cummax(lane_max)[L - 1]
            cand = jnp.where(lane_max == gm, lane_chunk * L + iota, D)
            idx = -plsc.cummax(-cand)[L - 1]
            idxs.append(idx)
            plsc.store_scatter(feat_l, [jnp.broadcast_to(row0 + idx, (L,))],
                               neg_inf, mask=mask0)
        a = (idxs[0] * D + idxs[1]) * D + idxs[2]
        b = idxs[3] * D + idxs[4]
        plsc.store_scatter(ka_l, [jnp.broadcast_to(r, (L,))],
                           jnp.broadcast_to(a, (L,)), mask=mask0)
        plsc.store_scatter(kb_l, [jnp.broadcast_to(r, (L,))],
                           jnp.broadcast_to(b, (L,)), mask=mask0)
        return _

    lax.fori_loop(0, RPW, row_body, 0)
    pltpu.sync_copy(ka_l, out_hbm.at[0, pl.ds(base, RPW)])
    pltpu.sync_copy(kb_l, out_hbm.at[1, pl.ds(base, RPW)])


def _tc_body(p_hbm, r_hbm, keys_hbm, out_ref,
             p_v, r_v, keys_v, sem_k, sem_p, sem_r):
    cp_k = pltpu.make_async_copy(keys_hbm, keys_v, sem_k)
    cp_p = pltpu.make_async_copy(p_hbm, p_v, sem_p)
    cp_r = pltpu.make_async_copy(r_hbm, r_v, sem_r)
    cp_k.start()
    cp_p.start()
    cp_r.start()
    cp_k.wait()
    cp_p.wait()
    cp_r.wait()

    ka = keys_v[0:1, :]
    kb = keys_v[1:2, :]
    r_all = r_v[:]
    acc = jnp.zeros((1, 1), jnp.float32)
    for k in range(NBLK):
        prod = jax.lax.dot_general(
            p_v[k * BLK:(k + 1) * BLK, :], r_all,
            (((1,), (1,)), ((), ())),
            preferred_element_type=jnp.float32)
        my_a = jnp.reshape(keys_v[0:1, k * BLK:(k + 1) * BLK], (BLK, 1))
        my_b = jnp.reshape(keys_v[1:2, k * BLK:(k + 1) * BLK], (BLK, 1))
        simb = (my_a == ka) & (my_b == kb)
        arg = jnp.where(simb, prod, 1.0 - prod)
        loss = jnp.maximum(jnp.log(arg), -100.0)
        acc += jnp.full((1, 1), -1.0 / (B * B)) * jnp.sum(loss)
    out_ref[:, :] = acc


@jax.jit
def kernel(unlabel_feat, unlabel_prob, rot_unlabel_prob):
    keys = _sc_keys(unlabel_feat.reshape(-1))
    out = pl.pallas_call(
        _tc_body,
        grid=(1,),
        in_specs=[
            pl.BlockSpec(memory_space=pl.ANY),
            pl.BlockSpec(memory_space=pl.ANY),
            pl.BlockSpec(memory_space=pl.ANY),
        ],
        out_specs=pl.BlockSpec((1, 1), lambda i: (0, 0)),
        out_shape=jax.ShapeDtypeStruct((1, 1), jnp.float32),
        scratch_shapes=[
            pltpu.VMEM((B, C), jnp.float32),
            pltpu.VMEM((B, C), jnp.float32),
            pltpu.VMEM((8, B), jnp.int32),
            pltpu.SemaphoreType.DMA,
            pltpu.SemaphoreType.DMA,
            pltpu.SemaphoreType.DMA,
        ],
    )(unlabel_prob, rot_unlabel_prob, keys)
    return out[0, 0]


# speedup vs baseline: 3.2106x; 3.2106x over previous
import jax
import jax.numpy as jnp
from jax.experimental import pallas as pl
from jax.experimental.pallas import tpu as pltpu

B = 1024
D = 512
C = 1000
K = 5
BLK = 128
NBLK = B // BLK


def _body(feat_hbm, p_hbm, r_hbm, out_ref,
          feat_v, p_v, r_v, keys_v, sem_f, sem_p, sem_r):
    cp_f = pltpu.make_async_copy(feat_hbm, feat_v, sem_f)
    cp_p = pltpu.make_async_copy(p_hbm, p_v, sem_p)
    cp_r = pltpu.make_async_copy(r_hbm, r_v, sem_r)
    cp_f.start()
    cp_p.start()
    cp_r.start()

    cp_f.wait()
    x = feat_v[:]
    iota = jax.lax.broadcasted_iota(jnp.int32, (B, D), 1)
    idxs = []
    for _ in range(K):
        m = jnp.max(x, axis=1, keepdims=True)
        idx = jnp.min(jnp.where(x == m, iota, D), axis=1)
        idxs.append(idx)
        x = jnp.where(iota == idx[:, None], -jnp.inf, x)
    a = (idxs[0] * D + idxs[1]) * D + idxs[2]
    b = idxs[3] * D + idxs[4]
    keys_v[:] = jnp.concatenate(
        [a[None, :], b[None, :], jnp.zeros((6, B), jnp.int32)], axis=0)

    cp_p.wait()
    cp_r.wait()

    ka = keys_v[0:1, :]
    kb = keys_v[1:2, :]
    prod = jax.lax.dot_general(
        p_v[:], r_v[:], (((1,), (1,)), ((), ())),
        preferred_element_type=jnp.float32)
    my_a = jnp.reshape(ka, (B, 1))
    my_b = jnp.reshape(kb, (B, 1))
    simb = (my_a == ka) & (my_b == kb)
    arg = jnp.where(simb, prod, 1.0 - prod)
    loss = jnp.maximum(jnp.log(arg), -100.0)
    out_ref[:, :] = jnp.full((1, 1), -1.0 / (B * B)) * jnp.sum(loss)


@jax.jit
def kernel(unlabel_feat, unlabel_prob, rot_unlabel_prob):
    out = pl.pallas_call(
        _body,
        grid=(1,),
        in_specs=[
            pl.BlockSpec(memory_space=pl.ANY),
            pl.BlockSpec(memory_space=pl.ANY),
            pl.BlockSpec(memory_space=pl.ANY),
        ],
        out_specs=pl.BlockSpec((1, 1), lambda i: (0, 0)),
        out_shape=jax.ShapeDtypeStruct((1, 1), jnp.float32),
        scratch_shapes=[
            pltpu.VMEM((B, D), jnp.float32),
            pltpu.VMEM((B, C), jnp.float32),
            pltpu.VMEM((B, C), jnp.float32),
            pltpu.VMEM((8, B), jnp.int32),
            pltpu.SemaphoreType.DMA,
            pltpu.SemaphoreType.DMA,
            pltpu.SemaphoreType.DMA,
        ],
    )(unlabel_feat, unlabel_prob, rot_unlabel_prob)
    return out[0, 0]


# speedup vs baseline: 3.2681x; 1.0179x over previous
import jax
import jax.numpy as jnp
from jax.experimental import pallas as pl
from jax.experimental.pallas import tpu as pltpu

B = 1024
D = 512
C = 1000
K = 5
BLK = 128
NBLK = B // BLK


def _body(feat_hbm, p_hbm, r_hbm, out_ref,
          feat_v, p_v, r_v, keys_v, sem_f, sem_p, sem_r):
    cp_f = pltpu.make_async_copy(feat_hbm, feat_v, sem_f)
    cp_p = pltpu.make_async_copy(p_hbm, p_v, sem_p)
    cp_r = pltpu.make_async_copy(r_hbm, r_v, sem_r)
    cp_f.start()
    cp_p.start()
    cp_r.start()

    cp_f.wait()
    x = feat_v[:]
    iota = jax.lax.broadcasted_iota(jnp.int32, (B, D), 1)
    idxs = []
    for _ in range(K):
        m = jnp.max(x, axis=1, keepdims=True)
        idx = jnp.min(jnp.where(x == m, iota, D), axis=1)
        idxs.append(idx)
        x = jnp.where(iota == idx[:, None], -jnp.inf, x)
    a = (idxs[0] * D + idxs[1]) * D + idxs[2]
    b = idxs[3] * D + idxs[4]
    keys_v[:] = jnp.concatenate(
        [a[None, :], b[None, :], jnp.zeros((6, B), jnp.int32)], axis=0)

    cp_p.wait()
    cp_r.wait()

    ka = keys_v[0:1, :]
    kb = keys_v[1:2, :]
    HBB = B // 2
    acc = jnp.zeros((1, 1), jnp.float32)
    for k in range(2):
        prod = jax.lax.dot_general(
            p_v[k * HBB:(k + 1) * HBB, :], r_v[:],
            (((1,), (1,)), ((), ())),
            preferred_element_type=jnp.float32)
        my_a = jnp.reshape(ka[:, k * HBB:(k + 1) * HBB], (HBB, 1))
        my_b = jnp.reshape(kb[:, k * HBB:(k + 1) * HBB], (HBB, 1))
        simb = (my_a == ka) & (my_b == kb)
        arg = jnp.where(simb, prod, 1.0 - prod)
        loss = jnp.maximum(jnp.log(arg), -100.0)
        acc += jnp.full((1, 1), -1.0 / (B * B)) * jnp.sum(loss)
    out_ref[:, :] = acc


@jax.jit
def kernel(unlabel_feat, unlabel_prob, rot_unlabel_prob):
    out = pl.pallas_call(
        _body,
        grid=(1,),
        in_specs=[
            pl.BlockSpec(memory_space=pl.ANY),
            pl.BlockSpec(memory_space=pl.ANY),
            pl.BlockSpec(memory_space=pl.ANY),
        ],
        out_specs=pl.BlockSpec((1, 1), lambda i: (0, 0)),
        out_shape=jax.ShapeDtypeStruct((1, 1), jnp.float32),
        scratch_shapes=[
            pltpu.VMEM((B, D), jnp.float32),
            pltpu.VMEM((B, C), jnp.float32),
            pltpu.VMEM((B, C), jnp.float32),
            pltpu.VMEM((8, B), jnp.int32),
            pltpu.SemaphoreType.DMA,
            pltpu.SemaphoreType.DMA,
            pltpu.SemaphoreType.DMA,
        ],
    )(unlabel_feat, unlabel_prob, rot_unlabel_prob)
    return out[0, 0]


# speedup vs baseline: 3.3209x; 1.0162x over previous
import jax
import jax.numpy as jnp
from jax.experimental import pallas as pl
from jax.experimental.pallas import tpu as pltpu

B = 1024
D = 512
C = 1000
K = 5
BLK = 128
NBLK = B // BLK


def _body(feat_hbm, p_hbm, r_hbm, out_ref,
          feat_v, p_v, r_v, keys_v, sem_f, sem_p, sem_r):
    cp_f = pltpu.make_async_copy(feat_hbm, feat_v, sem_f)
    cp_p = pltpu.make_async_copy(p_hbm, p_v, sem_p)
    cp_r = pltpu.make_async_copy(r_hbm, r_v, sem_r)
    cp_f.start()
    cp_p.start()
    cp_r.start()

    cp_f.wait()
    x = feat_v[:]
    iota = jax.lax.broadcasted_iota(jnp.int32, (B, D), 1)
    idxs = []
    for _ in range(K):
        m = jnp.max(x, axis=1, keepdims=True)
        idx = jnp.min(jnp.where(x == m, iota, D), axis=1)
        idxs.append(idx)
        x = jnp.where(iota == idx[:, None], -jnp.inf, x)
    a = (idxs[0] * D + idxs[1]) * D + idxs[2]
    b = idxs[3] * D + idxs[4]
    keys_v[:] = jnp.concatenate(
        [a[None, :], b[None, :], jnp.zeros((6, B), jnp.int32)], axis=0)

    cp_p.wait()
    cp_r.wait()

    ka = keys_v[0:1, :]
    kb = keys_v[1:2, :]
    HBB = B // 4
    acc = jnp.zeros((1, 1), jnp.float32)
    for k in range(4):
        prod = jax.lax.dot_general(
            p_v[k * HBB:(k + 1) * HBB, :], r_v[:],
            (((1,), (1,)), ((), ())),
            preferred_element_type=jnp.float32)
        my_a = jnp.reshape(ka[:, k * HBB:(k + 1) * HBB], (HBB, 1))
        my_b = jnp.reshape(kb[:, k * HBB:(k + 1) * HBB], (HBB, 1))
        simb = (my_a == ka) & (my_b == kb)
        arg = jnp.where(simb, prod, 1.0 - prod)
        loss = jnp.maximum(jnp.log(arg), -100.0)
        acc += jnp.full((1, 1), -1.0 / (B * B)) * jnp.sum(loss)
    out_ref[:, :] = acc


@jax.jit
def kernel(unlabel_feat, unlabel_prob, rot_unlabel_prob):
    out = pl.pallas_call(
        _body,
        grid=(1,),
        in_specs=[
            pl.BlockSpec(memory_space=pl.ANY),
            pl.BlockSpec(memory_space=pl.ANY),
            pl.BlockSpec(memory_space=pl.ANY),
        ],
        out_specs=pl.BlockSpec((1, 1), lambda i: (0, 0)),
        out_shape=jax.ShapeDtypeStruct((1, 1), jnp.float32),
        scratch_shapes=[
            pltpu.VMEM((B, D), jnp.float32),
            pltpu.VMEM((B, C), jnp.float32),
            pltpu.VMEM((B, C), jnp.float32),
            pltpu.VMEM((8, B), jnp.int32),
            pltpu.SemaphoreType.DMA,
            pltpu.SemaphoreType.DMA,
            pltpu.SemaphoreType.DMA,
        ],
    )(unlabel_feat, unlabel_prob, rot_unlabel_prob)
    return out[0, 0]
